# baseline (device time: 50626 ns/iter reference)
import jax
import jax.numpy as jnp
from jax import lax
from jax.experimental import pallas as pl
from jax.experimental.pallas import tpu as pltpu

T = 1024
D = 1024
F = 2048
E = 2
LCAP = 288
CAP = E * LCAP
NC = 6
CS = CAP // NC
QD = D // 4
QF = F // 4
NQ = 4


def kernel(x, assign, W1, W2):
    assign2d = assign.reshape(T, 1)

    def body(x_ref, a_ref, w1_hbm, w2_hbm,
             out_ref,
             xs_ref, xr_ref, ret_ref,
             st1a_ref, st1b_ref, st2a_ref, st2b_ref,
             w1b_ref, w2b_ref,
             send_sems, recv_sems, load_sems):
        my_x = lax.axis_index("x")
        my_yk = lax.axis_index("y")
        my_z = lax.axis_index("z")
        partner = (my_x, 1 - my_yk, my_z)

        jobs = []
        for e in range(E):
            for q in range(NQ):
                jobs.append((0, e, q))
                jobs.append((1, e, q))
        tensors = [
            (w1_hbm, w1b_ref, QD, (st1a_ref, st1b_ref)),
            (w2_hbm, w2b_ref, QF, (st2a_ref, st2b_ref)),
        ]
        cps = {}

        def start_job(j):
            tn, e, q = jobs[j]
            hbm, _, rows, bufs = tensors[tn]
            c = pltpu.make_async_copy(
                hbm.at[e, pl.ds(q * rows, rows)], bufs[q % 2],
                load_sems.at[j],
            )
            c.start()
            cps[j] = c

        def cast_job(j):
            tn, e, q = jobs[j]
            _, wb, rows, bufs = tensors[tn]
            cps[j].wait()
            wb[e, pl.ds(q * rows, rows)] = bufs[q % 2][...].astype(jnp.bfloat16)

        for j in range(4):
            start_job(j)

        barrier_sem = pltpu.get_barrier_semaphore()
        pl.semaphore_signal(barrier_sem, inc=1, device_id=partner,
                            device_id_type=pl.DeviceIdType.MESH)
        pl.semaphore_wait(barrier_sem, 1)

        a = a_ref[...]
        tri = (
            lax.broadcasted_iota(jnp.int32, (T, T), 0)
            >= lax.broadcasted_iota(jnp.int32, (T, T), 1)
        ).astype(jnp.bfloat16)
        onehot4 = (
            a == lax.broadcasted_iota(jnp.int32, (T, 4), 1)
        ).astype(jnp.bfloat16)
        cnt = jnp.dot(tri, onehot4, preferred_element_type=jnp.float32)
        rank = jnp.sum(onehot4.astype(jnp.float32) * (cnt - 1.0),
                       axis=1, keepdims=True).astype(jnp.int32)

        slot_pair = (a % 2) * LCAP + rank
        in_local = (a // 2) == my_yk
        lslot = jnp.where(in_local, slot_pair, CAP)
        sslot = jnp.where(in_local, CAP, slot_pair)

        iota = lax.broadcasted_iota(jnp.int32, (T, CAP), 1)
        sel_s = (iota == sslot).astype(jnp.bfloat16)
        sel_l = (iota == lslot).astype(jnp.bfloat16)

        xv = x_ref[...].astype(jnp.bfloat16)
        pack = lambda sel: lax.dot_general(
            sel, xv, (((0,), (0,)), ((), ())),
            preferred_element_type=jnp.float32,
        ).astype(jnp.bfloat16)

        xs_ref[...] = pack(sel_s)
        rdma_x = []
        for i in range(E):
            sl = pl.ds(i * LCAP, LCAP)
            r = pltpu.make_async_remote_copy(
                src_ref=xs_ref.at[sl], dst_ref=xr_ref.at[sl],
                send_sem=send_sems.at[i], recv_sem=recv_sems.at[i],
                device_id=partner, device_id_type=pl.DeviceIdType.MESH,
            )
            r.start()
            rdma_x.append(r)

        xpl = pack(sel_l)

        for j in range(4, 12):
            cast_job(j - 4)
            start_job(j)

        def ffn(xs, i):
            h = jnp.maximum(
                jnp.dot(xs, w1b_ref[i], preferred_element_type=jnp.float32),
                0.0,
            ).astype(jnp.bfloat16)
            return jnp.dot(h, w2b_ref[i], preferred_element_type=jnp.float32)

        ret_rdmas = []

        def remote_chunks(lo, hi):
            for c in range(lo, hi):
                sl = pl.ds(c * CS, CS)
                i = (c * CS) // LCAP
                xr_ref[sl, :] = ffn(xr_ref[sl, :], i).astype(jnp.bfloat16)
                r = pltpu.make_async_remote_copy(
                    src_ref=xr_ref.at[sl],
                    dst_ref=ret_ref.at[sl],
                    send_sem=send_sems.at[E + c],
                    recv_sem=recv_sems.at[E + c],
                    device_id=partner, device_id_type=pl.DeviceIdType.MESH,
                )
                r.start()
                ret_rdmas.append(r)

        rdma_x[0].wait()
        remote_chunks(0, NC // 2)

        for j in range(12, 16):
            cast_job(j - 4)
            start_job(j)
        for j in range(12, 16):
            cast_job(j)

        rdma_x[1].wait()
        remote_chunks(NC // 2, NC)

        o0 = ffn(xpl[:LCAP], 0).astype(jnp.bfloat16)
        o1 = ffn(xpl[LCAP:], 1).astype(jnp.bfloat16)
        out_ref[...] = jnp.dot(
            sel_l[:, :LCAP], o0, preferred_element_type=jnp.float32
        ) + jnp.dot(
            sel_l[:, LCAP:], o1, preferred_element_type=jnp.float32
        )

        half = CAP // 2
        for r in ret_rdmas[: NC // 2]:
            r.wait()
        out_ref[...] = out_ref[...] + jnp.dot(
            sel_s[:, :half], ret_ref[:half, :],
            preferred_element_type=jnp.float32,
        )
        for r in ret_rdmas[NC // 2:]:
            r.wait()
        out_ref[...] = out_ref[...] + jnp.dot(
            sel_s[:, half:], ret_ref[half:, :],
            preferred_element_type=jnp.float32,
        )

    return pl.pallas_call(
        body,
        out_shape=jax.ShapeDtypeStruct((T, D), jnp.float32),
        in_specs=[
            pl.BlockSpec(memory_space=pltpu.VMEM),
            pl.BlockSpec(memory_space=pltpu.VMEM),
            pl.BlockSpec(memory_space=pltpu.MemorySpace.HBM),
            pl.BlockSpec(memory_space=pltpu.MemorySpace.HBM),
        ],
        out_specs=pl.BlockSpec(memory_space=pltpu.VMEM),
        scratch_shapes=[
            pltpu.VMEM((CAP, D), jnp.bfloat16),
            pltpu.VMEM((CAP, D), jnp.bfloat16),
            pltpu.VMEM((CAP, D), jnp.bfloat16),
            pltpu.VMEM((QD, F), jnp.float32),
            pltpu.VMEM((QD, F), jnp.float32),
            pltpu.VMEM((QF, D), jnp.float32),
            pltpu.VMEM((QF, D), jnp.float32),
            pltpu.VMEM((E, D, F), jnp.bfloat16),
            pltpu.VMEM((E, F, D), jnp.bfloat16),
            pltpu.SemaphoreType.DMA((E + NC,)),
            pltpu.SemaphoreType.DMA((E + NC,)),
            pltpu.SemaphoreType.DMA((16,)),
        ],
        compiler_params=pltpu.CompilerParams(
            collective_id=0,
            vmem_limit_bytes=110 * 1024 * 1024,
        ),
    )(x, assign2d, W1, W2)


# device time: 47844 ns/iter; 1.0581x vs baseline; 1.0581x over previous
import jax
import jax.numpy as jnp
from jax import lax
from jax.experimental import pallas as pl
from jax.experimental.pallas import tpu as pltpu

T = 1024
D = 1024
F = 2048
E = 2
LCAP = 288
CAP = E * LCAP
NC = 4
CS = CAP // NC
HD = D // 2
HF = F // 2


def kernel(x, assign, W1, W2):
    assign2d = assign.reshape(T, 1)

    def body(x_ref, a_ref, w1_hbm, w2_hbm,
             out_ref,
             xs_ref, xr_ref, ret_ref,
             st1a_ref, st1b_ref, st2a_ref, st2b_ref,
             w1b_ref, w2b_ref,
             send_sems, recv_sems, load_sems):
        my_x = lax.axis_index("x")
        my_yk = lax.axis_index("y")
        my_z = lax.axis_index("z")
        partner = (my_x, 1 - my_yk, my_z)

        stages = [
            (w1_hbm, st1a_ref, HD, 0, w1b_ref),
            (w1_hbm, st1b_ref, HD, HD, w1b_ref),
            (w2_hbm, st2a_ref, HF, 0, w2b_ref),
            (w2_hbm, st2b_ref, HF, HF, w2b_ref),
        ]

        def start_load(i, k, s):
            hbm, st, rows, off, _ = stages[k]
            c = pltpu.make_async_copy(
                hbm.at[i, pl.ds(off, rows)], st, load_sems.at[s])
            c.start()
            return c

        def cast_load(i, k, c):
            _, st, rows, off, wb = stages[k]
            c.wait()
            wb[i, pl.ds(off, rows)] = st[...].astype(jnp.bfloat16)

        cs0 = [start_load(0, k, k) for k in range(4)]

        barrier_sem = pltpu.get_barrier_semaphore()
        pl.semaphore_signal(barrier_sem, inc=1, device_id=partner,
                            device_id_type=pl.DeviceIdType.MESH)
        pl.semaphore_wait(barrier_sem, 1)

        a = a_ref[...]
        tri = (
            lax.broadcasted_iota(jnp.int32, (T, T), 0)
            >= lax.broadcasted_iota(jnp.int32, (T, T), 1)
        ).astype(jnp.bfloat16)
        onehot4 = (
            a == lax.broadcasted_iota(jnp.int32, (T, 4), 1)
        ).astype(jnp.bfloat16)
        cnt = jnp.dot(tri, onehot4, preferred_element_type=jnp.float32)
        rank = jnp.sum(onehot4.astype(jnp.float32) * (cnt - 1.0),
                       axis=1, keepdims=True).astype(jnp.int32)

        slot_pair = (a % 2) * LCAP + rank
        in_local = (a // 2) == my_yk
        lslot = jnp.where(in_local, slot_pair, CAP)
        sslot = jnp.where(in_local, CAP, slot_pair)

        iota = lax.broadcasted_iota(jnp.int32, (T, CAP), 1)
        sel_s = (iota == sslot).astype(jnp.bfloat16)
        sel_l = (iota == lslot).astype(jnp.bfloat16)

        xv = x_ref[...].astype(jnp.bfloat16)
        pack = lambda sel: lax.dot_general(
            sel, xv, (((0,), (0,)), ((), ())),
            preferred_element_type=jnp.float32,
        ).astype(jnp.bfloat16)

        xs_ref[...] = pack(sel_s)
        rdma_x = []
        for i in range(E):
            sl = pl.ds(i * LCAP, LCAP)
            r = pltpu.make_async_remote_copy(
                src_ref=xs_ref.at[sl], dst_ref=xr_ref.at[sl],
                send_sem=send_sems.at[i], recv_sem=recv_sems.at[i],
                device_id=partner, device_id_type=pl.DeviceIdType.MESH,
            )
            r.start()
            rdma_x.append(r)

        xpl = pack(sel_l)

        cs1 = []
        for k in range(4):
            cast_load(0, k, cs0[k])
            cs1.append(start_load(1, k, 4 + k))

        def ffn(xs, i):
            h = jnp.maximum(
                jnp.dot(xs, w1b_ref[i], preferred_element_type=jnp.float32),
                0.0,
            ).astype(jnp.bfloat16)
            return jnp.dot(h, w2b_ref[i], preferred_element_type=jnp.float32)

        ret_rdmas = []

        def remote_chunks(lo, hi):
            for c in range(lo, hi):
                sl = pl.ds(c * CS, CS)
                i = (c * CS) // LCAP
                xr_ref[sl, :] = ffn(xr_ref[sl, :], i).astype(jnp.bfloat16)
                r = pltpu.make_async_remote_copy(
                    src_ref=xr_ref.at[sl],
                    dst_ref=ret_ref.at[sl],
                    send_sem=send_sems.at[E + c],
                    recv_sem=recv_sems.at[E + c],
                    device_id=partner, device_id_type=pl.DeviceIdType.MESH,
                )
                r.start()
                ret_rdmas.append(r)

        rdma_x[0].wait()
        remote_chunks(0, NC // 2)

        for k in range(4):
            cast_load(1, k, cs1[k])
        rdma_x[1].wait()
        remote_chunks(NC // 2, NC)

        o0 = ffn(xpl[:LCAP], 0).astype(jnp.bfloat16)
        o1 = ffn(xpl[LCAP:], 1).astype(jnp.bfloat16)
        out_ref[...] = jnp.dot(
            sel_l[:, :LCAP], o0, preferred_element_type=jnp.float32
        ) + jnp.dot(
            sel_l[:, LCAP:], o1, preferred_element_type=jnp.float32
        )

        half = CAP // 2
        for r in ret_rdmas[: NC // 2]:
            r.wait()
        out_ref[...] = out_ref[...] + jnp.dot(
            sel_s[:, :half], ret_ref[:half, :],
            preferred_element_type=jnp.float32,
        )
        for r in ret_rdmas[NC // 2:]:
            r.wait()
        out_ref[...] = out_ref[...] + jnp.dot(
            sel_s[:, half:], ret_ref[half:, :],
            preferred_element_type=jnp.float32,
        )

    return pl.pallas_call(
        body,
        out_shape=jax.ShapeDtypeStruct((T, D), jnp.float32),
        in_specs=[
            pl.BlockSpec(memory_space=pltpu.VMEM),
            pl.BlockSpec(memory_space=pltpu.VMEM),
            pl.BlockSpec(memory_space=pltpu.MemorySpace.HBM),
            pl.BlockSpec(memory_space=pltpu.MemorySpace.HBM),
        ],
        out_specs=pl.BlockSpec(memory_space=pltpu.VMEM),
        scratch_shapes=[
            pltpu.VMEM((CAP, D), jnp.bfloat16),
            pltpu.VMEM((CAP, D), jnp.bfloat16),
            pltpu.VMEM((CAP, D), jnp.bfloat16),
            pltpu.VMEM((HD, F), jnp.float32),
            pltpu.VMEM((HD, F), jnp.float32),
            pltpu.VMEM((HF, D), jnp.float32),
            pltpu.VMEM((HF, D), jnp.float32),
            pltpu.VMEM((E, D, F), jnp.bfloat16),
            pltpu.VMEM((E, F, D), jnp.bfloat16),
            pltpu.SemaphoreType.DMA((E + NC,)),
            pltpu.SemaphoreType.DMA((E + NC,)),
            pltpu.SemaphoreType.DMA((8,)),
        ],
        compiler_params=pltpu.CompilerParams(
            collective_id=0,
            vmem_limit_bytes=110 * 1024 * 1024,
        ),
    )(x, assign2d, W1, W2)
